# baseline (speedup 1.0000x reference)
import functools

import jax
import jax.numpy as jnp
from jax.experimental import pallas as pl
from jax.experimental.pallas import tpu as pltpu

_RG = 16
_MC = 1024


def _chamfer_body(
    x_hbm, y_hbm, loss_ref, xv_ref, yv_ref, yt_ref, ymin_ref, rmin_ref, sem,
    *, n_total, m_total, nt_steps, b_total, tn, m
):
    b = pl.program_id(0)
    nt = pl.program_id(1)
    inf = jnp.array(float("inf"), jnp.bfloat16)

    @pl.when(jnp.logical_and(b == 0, nt == 0))
    def _init_loss():
        loss_ref[0, 0] = 0.0

    cp = pltpu.make_async_copy(x_hbm.at[b], xv_ref, sem)
    cp.start()

    @pl.when(nt == 0)
    def _prep_y():
        cpy = pltpu.make_async_copy(y_hbm.at[b], yv_ref, sem)
        cpy.start()
        cpy.wait()
        yt_ref[...] = jnp.transpose(yv_ref[...]).astype(jnp.bfloat16)
        ymin_ref[...] = jnp.full((_RG, m), inf, jnp.bfloat16)

    cp.wait()
    x = xv_ref[...].astype(jnp.bfloat16)

    for mc in range(m // _MC):
        sl = slice(mc * _MC, (mc + 1) * _MC)
        y0 = yt_ref[0:1, sl]
        y1 = yt_ref[1:2, sl]
        y2 = yt_ref[2:3, sl]
        ym_acc = None
        for rg in range(tn // _RG):
            rs = slice(rg * _RG, (rg + 1) * _RG)
            xr = x[rs, :]
            d = (
                jnp.abs(xr[:, 0:1] - y0)
                + jnp.abs(xr[:, 1:2] - y1)
                + jnp.abs(xr[:, 2:3] - y2)
            )
            ym_acc = d if ym_acc is None else jnp.minimum(ym_acc, d)
            parts = [d[:, k * 128:(k + 1) * 128] for k in range(_MC // 128)]
            while len(parts) > 1:
                parts = [
                    jnp.minimum(parts[i], parts[i + 1])
                    for i in range(0, len(parts) - 1, 2)
                ] + ([parts[-1]] if len(parts) % 2 else [])
            dm = parts[0]
            if mc == 0:
                rmin_ref[rs, :] = dm
            else:
                rmin_ref[rs, :] = jnp.minimum(rmin_ref[rs, :], dm)
        ymin_ref[:, sl] = jnp.minimum(ymin_ref[:, sl], ym_acc)

    sx = jnp.sum(jnp.min(rmin_ref[...], axis=1).astype(jnp.float32))
    loss_ref[0, 0] += sx / (n_total * b_total)

    @pl.when(nt == nt_steps - 1)
    def _finish_batch():
        ys = jnp.sum(jnp.min(ymin_ref[...], axis=0).astype(jnp.float32))
        loss_ref[0, 0] += ys / (m_total * b_total)


def kernel(mesh_x, mesh_y):
    B, N, D = mesh_x.shape
    _, M, _ = mesh_y.shape
    TN = N
    NT = N // TN

    body = functools.partial(
        _chamfer_body,
        n_total=float(N),
        m_total=float(M),
        nt_steps=NT,
        b_total=float(B),
        tn=TN,
        m=M,
    )

    loss = pl.pallas_call(
        body,
        grid=(B, NT),
        in_specs=[
            pl.BlockSpec(memory_space=pltpu.MemorySpace.HBM),
            pl.BlockSpec(memory_space=pltpu.MemorySpace.HBM),
        ],
        out_specs=pl.BlockSpec(
            (1, 1), lambda b, nt: (0, 0), memory_space=pltpu.SMEM
        ),
        out_shape=jax.ShapeDtypeStruct((1, 1), jnp.float32),
        scratch_shapes=[
            pltpu.VMEM((TN, D), jnp.float32),
            pltpu.VMEM((M, D), jnp.float32),
            pltpu.VMEM((D, M), jnp.bfloat16),
            pltpu.VMEM((_RG, M), jnp.bfloat16),
            pltpu.VMEM((TN, 128), jnp.bfloat16),
            pltpu.SemaphoreType.DMA,
        ],
    )(mesh_x, mesh_y)

    return loss[0, 0]


# speedup vs baseline: 1.0393x; 1.0393x over previous
import functools

import jax
import jax.numpy as jnp
from jax.experimental import pallas as pl
from jax.experimental.pallas import tpu as pltpu

_RG = 16
_MC = 1024


def _chamfer_body(
    x_ref, y_ref, loss_ref, yt_ref, ymin_ref, rmin_ref,
    *, n_total, m_total, nt_steps, b_total, tn, m
):
    b = pl.program_id(0)
    nt = pl.program_id(1)
    inf = jnp.array(float("inf"), jnp.bfloat16)

    @pl.when(jnp.logical_and(b == 0, nt == 0))
    def _init_loss():
        loss_ref[0, 0] = 0.0

    @pl.when(nt == 0)
    def _prep_y():
        yt_ref[...] = jnp.transpose(y_ref[0]).astype(jnp.bfloat16)
        ymin_ref[...] = jnp.full((_RG, m), inf, jnp.bfloat16)

    x = x_ref[0].astype(jnp.bfloat16)

    for mc in range(m // _MC):
        sl = slice(mc * _MC, (mc + 1) * _MC)
        y0 = yt_ref[0:1, sl]
        y1 = yt_ref[1:2, sl]
        y2 = yt_ref[2:3, sl]
        ym_acc = None
        for rg in range(tn // _RG):
            rs = slice(rg * _RG, (rg + 1) * _RG)
            xr = x[rs, :]
            d = (
                jnp.abs(xr[:, 0:1] - y0)
                + jnp.abs(xr[:, 1:2] - y1)
                + jnp.abs(xr[:, 2:3] - y2)
            )
            ym_acc = d if ym_acc is None else jnp.minimum(ym_acc, d)
            parts = [d[:, k * 128:(k + 1) * 128] for k in range(_MC // 128)]
            while len(parts) > 1:
                parts = [
                    jnp.minimum(parts[i], parts[i + 1])
                    for i in range(0, len(parts) - 1, 2)
                ] + ([parts[-1]] if len(parts) % 2 else [])
            dm = parts[0]
            if mc == 0:
                rmin_ref[rs, :] = dm
            else:
                rmin_ref[rs, :] = jnp.minimum(rmin_ref[rs, :], dm)
        ymin_ref[:, sl] = jnp.minimum(ymin_ref[:, sl], ym_acc)

    sx = jnp.sum(jnp.min(rmin_ref[...], axis=1).astype(jnp.float32))
    loss_ref[0, 0] += sx / (n_total * b_total)

    @pl.when(nt == nt_steps - 1)
    def _finish_batch():
        ys = jnp.sum(jnp.min(ymin_ref[...], axis=0).astype(jnp.float32))
        loss_ref[0, 0] += ys / (m_total * b_total)


def kernel(mesh_x, mesh_y):
    B, N, D = mesh_x.shape
    _, M, _ = mesh_y.shape
    TN = N
    NT = N // TN

    body = functools.partial(
        _chamfer_body,
        n_total=float(N),
        m_total=float(M),
        nt_steps=NT,
        b_total=float(B),
        tn=TN,
        m=M,
    )

    loss = pl.pallas_call(
        body,
        grid=(B, NT),
        in_specs=[
            pl.BlockSpec((1, TN, D), lambda b, nt: (b, nt, 0)),
            pl.BlockSpec((1, M, D), lambda b, nt: (b, 0, 0)),
        ],
        out_specs=pl.BlockSpec(
            (1, 1), lambda b, nt: (0, 0), memory_space=pltpu.SMEM
        ),
        out_shape=jax.ShapeDtypeStruct((1, 1), jnp.float32),
        scratch_shapes=[
            pltpu.VMEM((D, M), jnp.bfloat16),
            pltpu.VMEM((_RG, M), jnp.bfloat16),
            pltpu.VMEM((TN, 128), jnp.bfloat16),
        ],
    )(mesh_x, mesh_y)

    return loss[0, 0]
